# Initial kernel scaffold; baseline (speedup 1.0000x reference)
import jax
import jax.numpy as jnp
from jax.experimental import pallas as pl


def kernel(ldesc0, ldesc1, line_enc0, line_enc1, lines_junc_idx0, lines_junc_idx1, W1, b1, gamma, beta, W2, b2):
    raise NotImplementedError("write your pallas kernel here")


# speedup vs baseline: 1.0000x; 1.0000x over previous
import jax
import jax.numpy as jnp
from jax.experimental import pallas as pl

EPS = 1e-5


def _mlp(x, W1, b1, gamma, beta, W2, b2):
    x = jnp.einsum('oc,bcl->bol', W1, x) + b1[None, :, None]
    mean = jnp.mean(x, axis=(0, 2), keepdims=True)
    var = jnp.mean((x - mean) ** 2, axis=(0, 2), keepdims=True)
    x = (x - mean) / jnp.sqrt(var + EPS)
    x = x * gamma[None, :, None] + beta[None, :, None]
    x = jax.nn.relu(x)
    x = jnp.einsum('oc,bcl->bol', W2, x) + b2[None, :, None]
    return x


def _endpoint_update(ldesc, line_enc, idx, W1, b1, gamma, beta, W2, b2):
    B, D, _ = ldesc.shape
    E = idx.shape[1]
    idx_b = jnp.broadcast_to(idx[:, None, :], (B, D, E))
    line_desc = jnp.take_along_axis(ldesc, idx_b, axis=2)
    flipped = line_desc.reshape(B, D, E // 2, 2)[..., ::-1].reshape(B, D, E)
    message = jnp.concatenate([line_desc, flipped, line_enc], axis=1)
    return _mlp(message, W1, b1, gamma, beta, W2, b2)


def _scatter_mean(idx, src, N):
    def per_batch(i, s):
        sums = jnp.zeros((s.shape[0], N), dtype=s.dtype).at[:, i].add(s)
        counts = jnp.zeros((N,), dtype=s.dtype).at[i].add(1.0)
        return jnp.where(counts[None, :] > 0, sums / jnp.maximum(counts, 1.0)[None, :], 0.0)
    return jax.vmap(per_batch)(idx, src)


def kernel(ldesc0, ldesc1, line_enc0, line_enc1, lines_junc_idx0, lines_junc_idx1, W1, b1, gamma, beta, W2, b2):
    N = ldesc0.shape[2]
    lupdate0 = _endpoint_update(ldesc0, line_enc0, lines_junc_idx0, W1, b1, gamma, beta, W2, b2)
    lupdate1 = _endpoint_update(ldesc1, line_enc1, lines_junc_idx1, W1, b1, gamma, beta, W2, b2)
    update0 = _scatter_mean(lines_junc_idx0, lupdate0, N)
    update1 = _scatter_mean(lines_junc_idx1, lupdate1, N)
    return (ldesc0 + update0, ldesc1 + update1)


# speedup vs baseline: 88.6839x; 88.6821x over previous
import functools

import jax
import jax.numpy as jnp
from jax import lax
from jax.experimental import pallas as pl
from jax.experimental.pallas import tpu as pltpu
from jax.experimental.pallas import tpu_sc as plsc

EPS = 1e-5
EBLK = 3200
_NC, _NS = 2, 16

_EPAD = 163840
_NPASS = 4
_OWN = 784
_NOUT = _NPASS * _NS * _OWN
_SEGE = 2000
_GB = 64


def _pairflip(v):
    n = v.shape[1]
    left = pltpu.roll(v, n - 1, axis=1)
    right = pltpu.roll(v, 1, axis=1)
    lane = lax.broadcasted_iota(jnp.int32, v.shape, 1)
    return jnp.where(lane % 2 == 0, left, right)


def _raw_x(g_blk, enc_blk, wa, wb, wc):
    f32 = jnp.float32
    u = lax.dot_general(wa, g_blk, (((1,), (1,)), ((), ())),
                        preferred_element_type=f32)
    v = lax.dot_general(wb, g_blk, (((1,), (1,)), ((), ())),
                        preferred_element_type=f32)
    w = lax.dot_general(wc, enc_blk, (((1,), (0,)), ((), ())),
                        preferred_element_type=f32)
    return u + _pairflip(v) + w


def _pass1_body(g_ref, enc_ref, wa_ref, wb_ref, wc_ref, b1_ref, s1_ref, s2_ref):
    x = _raw_x(g_ref[...], enc_ref[...], wa_ref[...], wb_ref[...], wc_ref[...])
    x = x + b1_ref[...]
    c = x.shape[0]
    p1 = jnp.sum(x.reshape(c, -1, 128), axis=1)
    p2 = jnp.sum((x * x).reshape(c, -1, 128), axis=1)

    @pl.when(pl.program_id(0) == 0)
    def _():
        s1_ref[...] = jnp.zeros_like(s1_ref)
        s2_ref[...] = jnp.zeros_like(s2_ref)

    s1_ref[...] += p1
    s2_ref[...] += p2


def _pass2_body(g_ref, enc_ref, wa_ref, wb_ref, wc_ref, scale_ref, shift_ref,
                w2_ref, b2_ref, out_ref):
    x = _raw_x(g_ref[...], enc_ref[...], wa_ref[...], wb_ref[...], wc_ref[...])
    r = jnp.maximum(scale_ref[...] * x + shift_ref[...], 0.0)
    y = lax.dot_general(r, w2_ref[...], (((0,), (1,)), ((), ())),
                        preferred_element_type=jnp.float32)
    out_ref[...] = y + b2_ref[...]


def _mlp_passes(g, enc, W1, b1, gamma, beta, W2, b2, interpret=False):
    D = g.shape[1]
    E = enc.shape[1]
    C = W1.shape[0]
    wa = W1[:, :D]
    wb = W1[:, D:2 * D]
    wc = W1[:, 2 * D:]
    nblk = E // EBLK

    full = lambda i: (0, 0)
    s1, s2 = pl.pallas_call(
        _pass1_body,
        grid=(nblk,),
        in_specs=[
            pl.BlockSpec((EBLK, D), lambda i: (i, 0)),
            pl.BlockSpec((D, EBLK), lambda i: (0, i)),
            pl.BlockSpec((C, D), full),
            pl.BlockSpec((C, D), full),
            pl.BlockSpec((C, D), full),
            pl.BlockSpec((C, 1), full),
        ],
        out_specs=[pl.BlockSpec((C, 128), full), pl.BlockSpec((C, 128), full)],
        out_shape=[jax.ShapeDtypeStruct((C, 128), jnp.float32)] * 2,
        compiler_params=pltpu.CompilerParams(
            dimension_semantics=("arbitrary",)),
        interpret=interpret,
    )(g, enc, wa, wb, wc, b1[:, None])

    mean = jnp.sum(s1, axis=1) / E
    ex2 = jnp.sum(s2, axis=1) / E
    var = ex2 - mean * mean
    rstd = lax.rsqrt(var + EPS)
    scale = gamma * rstd
    shift = scale * (b1 - mean) + beta

    y = pl.pallas_call(
        _pass2_body,
        grid=(nblk,),
        in_specs=[
            pl.BlockSpec((EBLK, D), lambda i: (i, 0)),
            pl.BlockSpec((D, EBLK), lambda i: (0, i)),
            pl.BlockSpec((C, D), full),
            pl.BlockSpec((C, D), full),
            pl.BlockSpec((C, D), full),
            pl.BlockSpec((C, 1), full),
            pl.BlockSpec((C, 1), full),
            pl.BlockSpec((D, C), full),
            pl.BlockSpec((1, D), full),
        ],
        out_specs=pl.BlockSpec((EBLK, D), lambda i: (i, 0)),
        out_shape=jax.ShapeDtypeStruct((_EPAD, D), jnp.float32),
        compiler_params=pltpu.CompilerParams(
            dimension_semantics=("arbitrary",)),
        interpret=interpret,
    )(g, enc, wa, wb, wc, scale[:, None], shift[:, None], W2, b2[None, :])
    return y


def _sc_gather(table, idxp):
    ep = idxp.shape[0]
    per = ep // (_NC * _NS)
    nb = per // 128
    mesh = plsc.VectorSubcoreMesh(core_axis_name="c", subcore_axis_name="s",
                                  num_cores=_NC, num_subcores=_NS)

    @functools.partial(
        pl.kernel, mesh=mesh,
        out_type=jax.ShapeDtypeStruct((ep, 128), jnp.float32),
        scratch_types=[
            pltpu.VMEM((per,), jnp.int32),
            pltpu.VMEM((128, 128), jnp.float32),
            pltpu.SemaphoreType.DMA,
        ],
        compiler_params=pltpu.CompilerParams(needs_layout_passes=False),
    )
    def k(table_hbm, idx_hbm, g_hbm, idx_v, rows_v, sem):
        wid = lax.axis_index("s") * _NC + lax.axis_index("c")
        base = wid * per
        pltpu.sync_copy(idx_hbm.at[pl.ds(base, per)], idx_v)

        def body(j, carry):
            sl = idx_v.at[pl.ds(j * 128, 128)]
            pltpu.async_copy(table_hbm.at[sl], rows_v, sem).wait()
            pltpu.sync_copy(rows_v, g_hbm.at[pl.ds(base + j * 128, 128)])
            return carry

        lax.fori_loop(0, nb, body, 0)

    return k(table, idxp)


def _sc_scatter(y0, y1, idxs0, idxsp1):
    e = idxs0.shape[0]
    nseg = e // _SEGE
    nvr = _SEGE // 16
    mesh = plsc.VectorSubcoreMesh(core_axis_name="c", subcore_axis_name="s",
                                  num_cores=_NC, num_subcores=_NS)

    @functools.partial(
        pl.kernel, mesh=mesh,
        out_type=[jax.ShapeDtypeStruct((_NOUT, 128), jnp.float32),
                  jax.ShapeDtypeStruct((_NOUT,), jnp.float32)] * 2,
        scratch_types=[
            pltpu.VMEM((_SEGE,), jnp.int32),
            pltpu.VMEM((_SEGE + 80,), jnp.int32),
            pltpu.VMEM((_SEGE + 80,), jnp.int32),
            pltpu.VMEM((_GB, 128), jnp.float32),
            pltpu.VMEM((_OWN + 8, 128), jnp.float32),
            pltpu.VMEM((_OWN + 16,), jnp.float32),
            pltpu.SemaphoreType.DMA,
        ],
        compiler_params=pltpu.CompilerParams(needs_layout_passes=False),
    )
    def k(y0_hbm, y1_hbm, i0_hbm, i1_hbm, os0_hbm, oc0_hbm, os1_hbm,
          oc1_hbm, idxb, eidb, locb, rowb, acc, cnt, sem):
        c = lax.axis_index("c")
        s = lax.axis_index("s")
        lane = lax.iota(jnp.int32, 16)
        zv = jnp.zeros((16,), jnp.float32)
        e0 = jnp.where(lane == 0, 1.0, 0.0).astype(jnp.float32)

        def run(y_hbm, i_hbm, os_hbm, oc_hbm):
            for p in range(_NPASS):
                own_base = p * _NS * _OWN + s * _OWN

                def zb(i, carry):
                    for u in range(8):
                        acc[i, pl.ds(u * 16, 16)] = zv
                    return carry
                lax.fori_loop(0, _OWN + 8, zb, 0)

                def zc(i, carry):
                    cnt[pl.ds(i * 16, 16)] = zv
                    return carry
                lax.fori_loop(0, (_OWN + 16) // 16, zc, 0)

                def seg_body(g, carry):
                    pltpu.sync_copy(i_hbm.at[pl.ds(g * _SEGE, _SEGE)], idxb)

                    def cb(i, kc):
                        v = idxb[pl.ds(i * 16, 16)]
                        rel = v - own_base
                        m = (rel >= 0) & (rel < _OWN)
                        mi = m.astype(jnp.int32)
                        pos = plsc.cumsum(mi) + (kc - 1)
                        eid = g * _SEGE + i * 16 + lane
                        plsc.store_scatter(eidb, [pos], eid, mask=m)
                        plsc.store_scatter(locb, [pos], rel, mask=m)
                        return kc + jnp.sum(mi)
                    kc = lax.fori_loop(0, nvr, cb, 0)

                    for q in range(_GB // 16):
                        pos = kc + q * 16 + lane
                        plsc.store_scatter(eidb, [pos],
                                           jnp.zeros((16,), jnp.int32))
                        plsc.store_scatter(locb, [pos],
                                           jnp.full((16,), _OWN, jnp.int32))

                    def gb(j, carry):
                        pltpu.async_copy(
                            y_hbm.at[eidb.at[pl.ds(j * _GB, _GB)]], rowb,
                            sem).wait()

                        def ab(r, carry2):
                            sl = locb[pl.ds(j * _GB + r, 16)][0]
                            for u in range(8):
                                acc[sl, pl.ds(u * 16, 16)] += \
                                    rowb[r, pl.ds(u * 16, 16)]
                            cnt[pl.ds(sl, 16)] += e0
                            return carry2
                        lax.fori_loop(0, _GB, ab, 0)
                        return carry
                    lax.fori_loop(0, (kc + _GB - 1) >> 6, gb, 0)
                    return carry
                lax.fori_loop(0, nseg, seg_body, 0)

                pltpu.sync_copy(acc.at[pl.ds(0, _OWN)],
                                os_hbm.at[pl.ds(own_base, _OWN)])
                pltpu.sync_copy(cnt.at[pl.ds(0, _OWN)],
                                oc_hbm.at[pl.ds(own_base, _OWN)])

        lax.cond(c == 0,
                 lambda: run(y0_hbm, i0_hbm, os0_hbm, oc0_hbm),
                 lambda: run(y1_hbm, i1_hbm, os1_hbm, oc1_hbm))

    return k(y0, y1, idxs0, idxsp1)


def _fin_body(sums_ref, cnt_ref, upd_ref):
    cnt = cnt_ref[...]
    recip = jnp.where(cnt > 0, 1.0 / jnp.maximum(cnt, 1.0), 0.0)
    upd_ref[...] = sums_ref[...] * recip


def _finalize(sums, cnt, n):
    nblk = 400
    return pl.pallas_call(
        _fin_body,
        grid=(n // nblk,),
        in_specs=[pl.BlockSpec((nblk, 128), lambda i: (i, 0)),
                  pl.BlockSpec((nblk, 1), lambda i: (i, 0))],
        out_specs=pl.BlockSpec((nblk, 128), lambda i: (i, 0)),
        out_shape=jax.ShapeDtypeStruct((n, 128), jnp.float32),
    )(sums, cnt)


def kernel(ldesc0, ldesc1, line_enc0, line_enc1, lines_junc_idx0,
           lines_junc_idx1, W1, b1, gamma, beta, W2, b2):
    N = ldesc0.shape[2]
    E = lines_junc_idx0.shape[1]

    def prep_idx(idx):
        pad = jnp.zeros((_EPAD - E,), jnp.int32)
        return jnp.concatenate([idx[0].astype(jnp.int32), pad])

    def front(ldesc, enc, idx):
        t = jnp.transpose(ldesc[0])
        g = _sc_gather(t, prep_idx(idx))
        return _mlp_passes(g, enc[0], W1, b1, gamma, beta, W2, b2)

    y0 = front(ldesc0, line_enc0, lines_junc_idx0)
    y1 = front(ldesc1, line_enc1, lines_junc_idx1)
    sums0, cnt0, sums1, cnt1 = _sc_scatter(
        y0, y1, lines_junc_idx0[0].astype(jnp.int32),
        lines_junc_idx1[0].astype(jnp.int32))
    upd0 = _finalize(sums0, cnt0[:, None], N)
    upd1 = _finalize(sums1, cnt1[:, None], N)
    return (ldesc0 + jnp.transpose(upd0)[None],
            ldesc1 + jnp.transpose(upd1)[None])


# speedup vs baseline: 162.5163x; 1.8325x over previous
import functools

import jax
import jax.numpy as jnp
from jax import lax
from jax.experimental import pallas as pl
from jax.experimental.pallas import tpu as pltpu
from jax.experimental.pallas import tpu_sc as plsc

EPS = 1e-5
EBLK = 3200
_NC, _NS = 2, 16

_EPAD = 163840
_NPASS = 4
_OWN = 784
_NOUT = _NPASS * _NS * _OWN
_SEGE = 4000
_GB = 64


def _pairflip(v):
    n = v.shape[1]
    left = pltpu.roll(v, n - 1, axis=1)
    right = pltpu.roll(v, 1, axis=1)
    lane = lax.broadcasted_iota(jnp.int32, v.shape, 1)
    return jnp.where(lane % 2 == 0, left, right)


def _raw_x(g_blk, enc_blk, wa, wb, wc):
    f32 = jnp.float32
    u = lax.dot_general(wa, g_blk, (((1,), (1,)), ((), ())),
                        preferred_element_type=f32)
    v = lax.dot_general(wb, g_blk, (((1,), (1,)), ((), ())),
                        preferred_element_type=f32)
    w = lax.dot_general(wc, enc_blk, (((1,), (0,)), ((), ())),
                        preferred_element_type=f32)
    return u + _pairflip(v) + w


def _pass1_body(g_ref, enc_ref, wa_ref, wb_ref, wc_ref, b1_ref, s1_ref, s2_ref):
    x = _raw_x(g_ref[...], enc_ref[...], wa_ref[...], wb_ref[...], wc_ref[...])
    x = x + b1_ref[...]
    c = x.shape[0]
    p1 = jnp.sum(x.reshape(c, -1, 128), axis=1)
    p2 = jnp.sum((x * x).reshape(c, -1, 128), axis=1)

    @pl.when(pl.program_id(0) == 0)
    def _():
        s1_ref[...] = jnp.zeros_like(s1_ref)
        s2_ref[...] = jnp.zeros_like(s2_ref)

    s1_ref[...] += p1
    s2_ref[...] += p2


def _pass2_body(g_ref, enc_ref, wa_ref, wb_ref, wc_ref, scale_ref, shift_ref,
                w2_ref, b2_ref, out_ref):
    x = _raw_x(g_ref[...], enc_ref[...], wa_ref[...], wb_ref[...], wc_ref[...])
    r = jnp.maximum(scale_ref[...] * x + shift_ref[...], 0.0)
    y = lax.dot_general(r, w2_ref[...], (((0,), (1,)), ((), ())),
                        preferred_element_type=jnp.float32)
    out_ref[...] = y + b2_ref[...]


def _mlp_passes(g, enc, W1, b1, gamma, beta, W2, b2, interpret=False):
    D = g.shape[1]
    E = enc.shape[1]
    C = W1.shape[0]
    wa = W1[:, :D]
    wb = W1[:, D:2 * D]
    wc = W1[:, 2 * D:]
    nblk = E // EBLK

    full = lambda i: (0, 0)
    s1, s2 = pl.pallas_call(
        _pass1_body,
        grid=(nblk,),
        in_specs=[
            pl.BlockSpec((EBLK, D), lambda i: (i, 0)),
            pl.BlockSpec((D, EBLK), lambda i: (0, i)),
            pl.BlockSpec((C, D), full),
            pl.BlockSpec((C, D), full),
            pl.BlockSpec((C, D), full),
            pl.BlockSpec((C, 1), full),
        ],
        out_specs=[pl.BlockSpec((C, 128), full), pl.BlockSpec((C, 128), full)],
        out_shape=[jax.ShapeDtypeStruct((C, 128), jnp.float32)] * 2,
        compiler_params=pltpu.CompilerParams(
            dimension_semantics=("arbitrary",)),
        interpret=interpret,
    )(g, enc, wa, wb, wc, b1[:, None])

    mean = jnp.sum(s1, axis=1) / E
    ex2 = jnp.sum(s2, axis=1) / E
    var = ex2 - mean * mean
    rstd = lax.rsqrt(var + EPS)
    scale = gamma * rstd
    shift = scale * (b1 - mean) + beta

    y = pl.pallas_call(
        _pass2_body,
        grid=(nblk,),
        in_specs=[
            pl.BlockSpec((EBLK, D), lambda i: (i, 0)),
            pl.BlockSpec((D, EBLK), lambda i: (0, i)),
            pl.BlockSpec((C, D), full),
            pl.BlockSpec((C, D), full),
            pl.BlockSpec((C, D), full),
            pl.BlockSpec((C, 1), full),
            pl.BlockSpec((C, 1), full),
            pl.BlockSpec((D, C), full),
            pl.BlockSpec((1, D), full),
        ],
        out_specs=pl.BlockSpec((EBLK, D), lambda i: (i, 0)),
        out_shape=jax.ShapeDtypeStruct((_EPAD, D), jnp.float32),
        compiler_params=pltpu.CompilerParams(
            dimension_semantics=("arbitrary",)),
        interpret=interpret,
    )(g, enc, wa, wb, wc, scale[:, None], shift[:, None], W2, b2[None, :])
    return y


def _sc_gather(table, idxp):
    ep = idxp.shape[0]
    per = ep // (_NC * _NS)
    nb = per // 128
    mesh = plsc.VectorSubcoreMesh(core_axis_name="c", subcore_axis_name="s",
                                  num_cores=_NC, num_subcores=_NS)

    @functools.partial(
        pl.kernel, mesh=mesh,
        out_type=jax.ShapeDtypeStruct((ep, 128), jnp.float32),
        scratch_types=[
            pltpu.VMEM((per,), jnp.int32),
            pltpu.VMEM((128, 128), jnp.float32),
            pltpu.SemaphoreType.DMA,
        ],
        compiler_params=pltpu.CompilerParams(needs_layout_passes=False),
    )
    def k(table_hbm, idx_hbm, g_hbm, idx_v, rows_v, sem):
        wid = lax.axis_index("s") * _NC + lax.axis_index("c")
        base = wid * per
        pltpu.sync_copy(idx_hbm.at[pl.ds(base, per)], idx_v)

        def body(j, carry):
            sl = idx_v.at[pl.ds(j * 128, 128)]
            pltpu.async_copy(table_hbm.at[sl], rows_v, sem).wait()
            pltpu.sync_copy(rows_v, g_hbm.at[pl.ds(base + j * 128, 128)])
            return carry

        lax.fori_loop(0, nb, body, 0)

    return k(table, idxp)


def _sc_scatter(y0, y1, idxs0, idxsp1):
    e = idxs0.shape[0]
    nseg = e // _SEGE
    nvr = _SEGE // 16
    mesh = plsc.VectorSubcoreMesh(core_axis_name="c", subcore_axis_name="s",
                                  num_cores=_NC, num_subcores=_NS)

    @functools.partial(
        pl.kernel, mesh=mesh,
        out_type=[jax.ShapeDtypeStruct((_NOUT, 128), jnp.float32),
                  jax.ShapeDtypeStruct((_NOUT,), jnp.float32)] * 2,
        scratch_types=[
            pltpu.VMEM((2 * _SEGE,), jnp.int32),
            pltpu.VMEM((_SEGE + 80,), jnp.int32),
            pltpu.VMEM((_SEGE + 80,), jnp.int32),
            pltpu.VMEM((_GB, 128), jnp.float32),
            pltpu.VMEM((_OWN + 8, 128), jnp.float32),
            pltpu.VMEM((_OWN + 16,), jnp.float32),
            pltpu.SemaphoreType.DMA,
            pltpu.SemaphoreType.DMA,
        ],
        compiler_params=pltpu.CompilerParams(needs_layout_passes=False),
    )
    def k(y0_hbm, y1_hbm, i0_hbm, i1_hbm, os0_hbm, oc0_hbm, os1_hbm,
          oc1_hbm, idxb, eidb, locb, rowb, acc, cnt, sem, isem):
        c = lax.axis_index("c")
        s = lax.axis_index("s")
        lane = lax.iota(jnp.int32, 16)
        zv = jnp.zeros((16,), jnp.float32)
        e0 = jnp.where(lane == 0, 1.0, 0.0).astype(jnp.float32)

        def run(y_hbm, i_hbm, os_hbm, oc_hbm):
            for p in range(_NPASS):
                own_base = p * _NS * _OWN + s * _OWN

                def zb(i, carry):
                    for u in range(8):
                        acc[i, pl.ds(u * 16, 16)] = zv
                    return carry
                lax.fori_loop(0, _OWN + 8, zb, 0)

                def zc(i, carry):
                    cnt[pl.ds(i * 16, 16)] = zv
                    return carry
                lax.fori_loop(0, (_OWN + 16) // 16, zc, 0)

                pltpu.async_copy(i_hbm.at[pl.ds(0, _SEGE)],
                                 idxb.at[pl.ds(0, _SEGE)], isem).wait()

                def seg_body(g, carry):
                    par = g & 1
                    nxt = lax.cond(g + 1 < nseg, lambda: g + 1, lambda: g)
                    cp = pltpu.async_copy(
                        i_hbm.at[pl.ds(nxt * _SEGE, _SEGE)],
                        idxb.at[pl.ds((1 - par) * _SEGE, _SEGE)], isem)

                    def cb(i, kc):
                        v = idxb[pl.ds(par * _SEGE + i * 16, 16)]
                        rel = v - own_base
                        m = (rel >= 0) & (rel < _OWN)
                        mi = m.astype(jnp.int32)
                        pos = plsc.cumsum(mi) + (kc - 1)
                        eid = g * _SEGE + i * 16 + lane
                        plsc.store_scatter(eidb, [pos], eid, mask=m)
                        plsc.store_scatter(locb, [pos], rel, mask=m)
                        return kc + jnp.sum(mi)
                    kc = lax.fori_loop(0, nvr, cb, 0)

                    for q in range(_GB // 16):
                        pos = kc + q * 16 + lane
                        plsc.store_scatter(eidb, [pos],
                                           jnp.zeros((16,), jnp.int32))
                        plsc.store_scatter(locb, [pos],
                                           jnp.full((16,), _OWN, jnp.int32))

                    def gb(j, carry):
                        pltpu.async_copy(
                            y_hbm.at[eidb.at[pl.ds(j * _GB, _GB)]], rowb,
                            sem).wait()

                        def ab(r, carry2):
                            sl = locb[pl.ds(j * _GB + r, 16)][0]
                            for u in range(8):
                                acc[sl, pl.ds(u * 16, 16)] += \
                                    rowb[r, pl.ds(u * 16, 16)]
                            cnt[pl.ds(sl, 16)] += e0
                            return carry2
                        lax.fori_loop(0, _GB, ab, 0)
                        return carry
                    lax.fori_loop(0, (kc + _GB - 1) >> 6, gb, 0)
                    cp.wait()
                    return carry
                lax.fori_loop(0, nseg, seg_body, 0)

                pltpu.sync_copy(acc.at[pl.ds(0, _OWN)],
                                os_hbm.at[pl.ds(own_base, _OWN)])
                pltpu.sync_copy(cnt.at[pl.ds(0, _OWN)],
                                oc_hbm.at[pl.ds(own_base, _OWN)])

        lax.cond(c == 0,
                 lambda: run(y0_hbm, i0_hbm, os0_hbm, oc0_hbm),
                 lambda: run(y1_hbm, i1_hbm, os1_hbm, oc1_hbm))

    return k(y0, y1, idxs0, idxsp1)


def _fin_body(sums_ref, cnt_ref, upd_ref):
    cnt = cnt_ref[...]
    recip = jnp.where(cnt > 0, 1.0 / jnp.maximum(cnt, 1.0), 0.0)
    upd_ref[...] = sums_ref[...] * recip


def _finalize(sums, cnt, n):
    nblk = 400
    return pl.pallas_call(
        _fin_body,
        grid=(n // nblk,),
        in_specs=[pl.BlockSpec((nblk, 128), lambda i: (i, 0)),
                  pl.BlockSpec((nblk, 1), lambda i: (i, 0))],
        out_specs=pl.BlockSpec((nblk, 128), lambda i: (i, 0)),
        out_shape=jax.ShapeDtypeStruct((n, 128), jnp.float32),
    )(sums, cnt)


def kernel(ldesc0, ldesc1, line_enc0, line_enc1, lines_junc_idx0,
           lines_junc_idx1, W1, b1, gamma, beta, W2, b2):
    N = ldesc0.shape[2]
    E = lines_junc_idx0.shape[1]

    def prep_idx(idx):
        pad = jnp.zeros((_EPAD - E,), jnp.int32)
        return jnp.concatenate([idx[0].astype(jnp.int32), pad])

    def front(ldesc, enc, idx):
        t = jnp.transpose(ldesc[0])
        g = _sc_gather(t, prep_idx(idx))
        return _mlp_passes(g, enc[0], W1, b1, gamma, beta, W2, b2)

    y0 = front(ldesc0, line_enc0, lines_junc_idx0)
    y1 = front(ldesc1, line_enc1, lines_junc_idx1)
    sums0, cnt0, sums1, cnt1 = _sc_scatter(
        y0, y1, lines_junc_idx0[0].astype(jnp.int32),
        lines_junc_idx1[0].astype(jnp.int32))
    upd0 = _finalize(sums0, cnt0[:, None], N)
    upd1 = _finalize(sums1, cnt1[:, None], N)
    return (ldesc0 + jnp.transpose(upd0)[None],
            ldesc1 + jnp.transpose(upd1)[None])


# speedup vs baseline: 162.6203x; 1.0006x over previous
import functools

import jax
import jax.numpy as jnp
from jax import lax
from jax.experimental import pallas as pl
from jax.experimental.pallas import tpu as pltpu
from jax.experimental.pallas import tpu_sc as plsc

EPS = 1e-5
EBLK = 3200
_NC, _NS = 2, 16

_EPAD = 163840
_NPASS = 4
_OWN = 784
_NOUT = _NPASS * _NS * _OWN
_SEGE = 4000
_GB = 64


def _pairflip(v):
    n = v.shape[1]
    left = pltpu.roll(v, n - 1, axis=1)
    right = pltpu.roll(v, 1, axis=1)
    lane = lax.broadcasted_iota(jnp.int32, v.shape, 1)
    return jnp.where(lane % 2 == 0, left, right)


def _raw_x(g_blk, enc_blk, wa, wb, wc):
    f32 = jnp.float32
    u = lax.dot_general(wa, g_blk, (((1,), (1,)), ((), ())),
                        preferred_element_type=f32)
    v = lax.dot_general(wb, g_blk, (((1,), (1,)), ((), ())),
                        preferred_element_type=f32)
    w = lax.dot_general(wc, enc_blk, (((1,), (0,)), ((), ())),
                        preferred_element_type=f32)
    return u + _pairflip(v) + w


def _pass1_body(g_ref, enc_ref, wa_ref, wb_ref, wc_ref, b1_ref, s1_ref, s2_ref):
    x = _raw_x(g_ref[...], enc_ref[...], wa_ref[...], wb_ref[...], wc_ref[...])
    x = x + b1_ref[...]
    c = x.shape[0]
    p1 = jnp.sum(x.reshape(c, -1, 128), axis=1)
    p2 = jnp.sum((x * x).reshape(c, -1, 128), axis=1)

    @pl.when(pl.program_id(0) == 0)
    def _():
        s1_ref[...] = jnp.zeros_like(s1_ref)
        s2_ref[...] = jnp.zeros_like(s2_ref)

    s1_ref[...] += p1
    s2_ref[...] += p2


def _pass2_body(g_ref, enc_ref, wa_ref, wb_ref, wc_ref, scale_ref, shift_ref,
                w2_ref, b2_ref, out_ref):
    x = _raw_x(g_ref[...], enc_ref[...], wa_ref[...], wb_ref[...], wc_ref[...])
    r = jnp.maximum(scale_ref[...] * x + shift_ref[...], 0.0)
    y = lax.dot_general(r, w2_ref[...], (((0,), (1,)), ((), ())),
                        preferred_element_type=jnp.float32)
    out_ref[...] = y + b2_ref[...]


def _mlp_passes(g, enc, W1, b1, gamma, beta, W2, b2, interpret=False):
    D = g.shape[1]
    E = enc.shape[1]
    C = W1.shape[0]
    wa = W1[:, :D]
    wb = W1[:, D:2 * D]
    wc = W1[:, 2 * D:]
    nblk = E // EBLK

    full = lambda i: (0, 0)
    s1, s2 = pl.pallas_call(
        _pass1_body,
        grid=(nblk,),
        in_specs=[
            pl.BlockSpec((EBLK, D), lambda i: (i, 0)),
            pl.BlockSpec((D, EBLK), lambda i: (0, i)),
            pl.BlockSpec((C, D), full),
            pl.BlockSpec((C, D), full),
            pl.BlockSpec((C, D), full),
            pl.BlockSpec((C, 1), full),
        ],
        out_specs=[pl.BlockSpec((C, 128), full), pl.BlockSpec((C, 128), full)],
        out_shape=[jax.ShapeDtypeStruct((C, 128), jnp.float32)] * 2,
        compiler_params=pltpu.CompilerParams(
            dimension_semantics=("arbitrary",)),
        interpret=interpret,
    )(g, enc, wa, wb, wc, b1[:, None])

    mean = jnp.sum(s1, axis=1) / E
    ex2 = jnp.sum(s2, axis=1) / E
    var = ex2 - mean * mean
    rstd = lax.rsqrt(var + EPS)
    scale = gamma * rstd
    shift = scale * (b1 - mean) + beta

    y = pl.pallas_call(
        _pass2_body,
        grid=(nblk,),
        in_specs=[
            pl.BlockSpec((EBLK, D), lambda i: (i, 0)),
            pl.BlockSpec((D, EBLK), lambda i: (0, i)),
            pl.BlockSpec((C, D), full),
            pl.BlockSpec((C, D), full),
            pl.BlockSpec((C, D), full),
            pl.BlockSpec((C, 1), full),
            pl.BlockSpec((C, 1), full),
            pl.BlockSpec((D, C), full),
            pl.BlockSpec((1, D), full),
        ],
        out_specs=pl.BlockSpec((EBLK, D), lambda i: (i, 0)),
        out_shape=jax.ShapeDtypeStruct((_EPAD, D), jnp.float32),
        compiler_params=pltpu.CompilerParams(
            dimension_semantics=("arbitrary",)),
        interpret=interpret,
    )(g, enc, wa, wb, wc, scale[:, None], shift[:, None], W2, b2[None, :])
    return y


def _sc_gather(table, idxp):
    ep = idxp.shape[0]
    per = ep // (_NC * _NS)
    nb = per // 128
    mesh = plsc.VectorSubcoreMesh(core_axis_name="c", subcore_axis_name="s",
                                  num_cores=_NC, num_subcores=_NS)

    @functools.partial(
        pl.kernel, mesh=mesh,
        out_type=jax.ShapeDtypeStruct((ep, 128), jnp.float32),
        scratch_types=[
            pltpu.VMEM((per,), jnp.int32),
            pltpu.VMEM((128, 128), jnp.float32),
            pltpu.SemaphoreType.DMA,
        ],
        compiler_params=pltpu.CompilerParams(needs_layout_passes=False),
    )
    def k(table_hbm, idx_hbm, g_hbm, idx_v, rows_v, sem):
        wid = lax.axis_index("s") * _NC + lax.axis_index("c")
        base = wid * per
        pltpu.sync_copy(idx_hbm.at[pl.ds(base, per)], idx_v)

        def body(j, carry):
            sl = idx_v.at[pl.ds(j * 128, 128)]
            pltpu.async_copy(table_hbm.at[sl], rows_v, sem).wait()
            pltpu.sync_copy(rows_v, g_hbm.at[pl.ds(base + j * 128, 128)])
            return carry

        lax.fori_loop(0, nb, body, 0)

    return k(table, idxp)


def _sc_scatter(y0, y1, idxs0, idxs1):
    e = idxs0.shape[0]
    nseg = e // _SEGE
    nvr = _SEGE // 16
    mesh = plsc.VectorSubcoreMesh(core_axis_name="c", subcore_axis_name="s",
                                  num_cores=_NC, num_subcores=_NS)

    @functools.partial(
        pl.kernel, mesh=mesh,
        out_type=[jax.ShapeDtypeStruct((_NOUT, 128), jnp.float32),
                  jax.ShapeDtypeStruct((_NOUT,), jnp.float32)] * 2,
        scratch_types=[
            pltpu.VMEM((2 * _SEGE,), jnp.int32),
            pltpu.VMEM((_SEGE + 80,), jnp.int32),
            pltpu.VMEM((_SEGE + 80,), jnp.int32),
            pltpu.VMEM((_GB, 128), jnp.float32),
            pltpu.VMEM((_OWN + 8, 128), jnp.float32),
            pltpu.VMEM((_OWN + 16,), jnp.float32),
            pltpu.SemaphoreType.DMA,
            pltpu.SemaphoreType.DMA,
        ],
        compiler_params=pltpu.CompilerParams(needs_layout_passes=False),
    )
    def k(y0_hbm, y1_hbm, i0_hbm, i1_hbm, os0_hbm, oc0_hbm, os1_hbm,
          oc1_hbm, idxb, eidb, locb, rowb, acc, cnt, sem, isem):
        c = lax.axis_index("c")
        s = lax.axis_index("s")
        lane = lax.iota(jnp.int32, 16)
        zv = jnp.zeros((16,), jnp.float32)
        e0 = jnp.where(lane == 0, 1.0, 0.0).astype(jnp.float32)

        def run(y_hbm, i_hbm, os_hbm, oc_hbm):
            for p in range(_NPASS):
                own_base = p * _NS * _OWN + s * _OWN

                def zb(i, carry):
                    for u in range(8):
                        acc[i, pl.ds(u * 16, 16)] = zv
                    return carry
                lax.fori_loop(0, _OWN + 8, zb, 0)

                def zc(i, carry):
                    cnt[pl.ds(i * 16, 16)] = zv
                    return carry
                lax.fori_loop(0, (_OWN + 16) // 16, zc, 0)

                pltpu.async_copy(i_hbm.at[pl.ds(0, _SEGE)],
                                 idxb.at[pl.ds(0, _SEGE)], isem).wait()

                def seg_body(g, carry):
                    par = g & 1
                    nxt = lax.cond(g + 1 < nseg, lambda: g + 1, lambda: g)
                    cp = pltpu.async_copy(
                        i_hbm.at[pl.ds(nxt * _SEGE, _SEGE)],
                        idxb.at[pl.ds((1 - par) * _SEGE, _SEGE)], isem)

                    def cb(i, kc):
                        v = idxb[pl.ds(par * _SEGE + i * 16, 16)]
                        rel = v - own_base
                        m = (rel >= 0) & (rel < _OWN)
                        mi = m.astype(jnp.int32)
                        pos = plsc.cumsum(mi) + (kc - 1)
                        eid = g * _SEGE + i * 16 + lane
                        plsc.store_scatter(eidb, [pos], eid, mask=m)
                        plsc.store_scatter(locb, [pos], rel, mask=m)
                        return kc + jnp.sum(mi)
                    kc = lax.fori_loop(0, nvr, cb, 0)

                    for q in range(_GB // 16):
                        pos = kc + q * 16 + lane
                        plsc.store_scatter(eidb, [pos],
                                           jnp.zeros((16,), jnp.int32))
                        plsc.store_scatter(locb, [pos],
                                           jnp.full((16,), _OWN, jnp.int32))

                    def gb(j, carry):
                        pltpu.async_copy(
                            y_hbm.at[eidb.at[pl.ds(j * _GB, _GB)]], rowb,
                            sem).wait()

                        def ab(r, carry2):
                            sl = locb[pl.ds(j * _GB + r, 16)][0]
                            for u in range(8):
                                acc[sl, pl.ds(u * 16, 16)] += \
                                    rowb[r, pl.ds(u * 16, 16)]
                            cnt[pl.ds(sl, 16)] += e0
                            return carry2
                        lax.fori_loop(0, _GB, ab, 0)
                        return carry
                    lax.fori_loop(0, (kc + _GB - 1) >> 6, gb, 0)
                    cp.wait()
                    return carry
                lax.fori_loop(0, nseg, seg_body, 0)

                pltpu.sync_copy(acc.at[pl.ds(0, _OWN)],
                                os_hbm.at[pl.ds(own_base, _OWN)])
                pltpu.sync_copy(cnt.at[pl.ds(0, _OWN)],
                                oc_hbm.at[pl.ds(own_base, _OWN)])

        lax.cond(c == 0,
                 lambda: run(y0_hbm, i0_hbm, os0_hbm, oc0_hbm),
                 lambda: run(y1_hbm, i1_hbm, os1_hbm, oc1_hbm))

    return k(y0, y1, idxs0, idxs1)


def _fin_body(sums_ref, cnt_ref, upd_ref):
    cnt = cnt_ref[...]
    recip = jnp.where(cnt > 0, 1.0 / jnp.maximum(cnt, 1.0), 0.0)
    upd_ref[...] = sums_ref[...] * recip


def _finalize(sums, cnt, n):
    nblk = 400
    return pl.pallas_call(
        _fin_body,
        grid=(n // nblk,),
        in_specs=[pl.BlockSpec((nblk, 128), lambda i: (i, 0)),
                  pl.BlockSpec((nblk, 1), lambda i: (i, 0))],
        out_specs=pl.BlockSpec((nblk, 128), lambda i: (i, 0)),
        out_shape=jax.ShapeDtypeStruct((n, 128), jnp.float32),
    )(sums, cnt)


def kernel(ldesc0, ldesc1, line_enc0, line_enc1, lines_junc_idx0,
           lines_junc_idx1, W1, b1, gamma, beta, W2, b2):
    N = ldesc0.shape[2]
    E = lines_junc_idx0.shape[1]

    def prep_idx(idx):
        pad = jnp.zeros((_EPAD - E,), jnp.int32)
        return jnp.concatenate([idx[0].astype(jnp.int32), pad])

    def front(ldesc, enc, idx):
        t = jnp.transpose(ldesc[0])
        g = _sc_gather(t, prep_idx(idx))
        return _mlp_passes(g, enc[0], W1, b1, gamma, beta, W2, b2)

    y0 = front(ldesc0, line_enc0, lines_junc_idx0)
    y1 = front(ldesc1, line_enc1, lines_junc_idx1)
    sums0, cnt0, sums1, cnt1 = _sc_scatter(
        y0, y1, lines_junc_idx0[0].astype(jnp.int32),
        lines_junc_idx1[0].astype(jnp.int32))
    upd0 = _finalize(sums0, cnt0[:, None], N)
    upd1 = _finalize(sums1, cnt1[:, None], N)
    return (ldesc0 + jnp.transpose(upd0)[None],
            ldesc1 + jnp.transpose(upd1)[None])
